# baseline (speedup 1.0000x reference)
import functools

import jax
import jax.numpy as jnp
from jax import lax
from jax.experimental import pallas as pl
from jax.experimental.pallas import tpu as pltpu
from jax.experimental.pallas import tpu_sc as plsc

V, D, B, L = 1000, 64, 4096, 200
VP = 1024
NC, NS = 2, 16
NW = NC * NS
RPW = 2 * B // NW
RC = 64
NCHUNK = RPW // RC
BB = 256
BM = 1024
BN = 1024


def _hist_kernel(idx_hbm, counts_hbm, idx_v, hist_v):
    wid = lax.axis_index("s") * NC + lax.axis_index("c")
    lane = lax.iota(jnp.int32, 16)
    ones = jnp.ones((16,), jnp.float32)
    zeros = jnp.zeros((16,), jnp.float32)

    def scatter(value, add):
        def lbody(l, c):
            lsplat = jnp.full((16,), 0, jnp.int32) + l
            for g in range(RC // 16):
                rows = lane + (g * 16)
                iv = plsc.load_gather(idx_v, [rows, lsplat])
                if add:
                    plsc.addupdate_scatter(hist_v, [rows, iv], value)
                else:
                    plsc.store_scatter(hist_v, [rows, iv], value)
            return c

        lax.fori_loop(0, L, lbody, 0)

    def zero_body(r, c):
        for u in range(VP // 16):
            hist_v[r, pl.ds(u * 16, 16)] = zeros
        return c

    lax.fori_loop(0, RC, zero_body, 0)
    for chunk in range(NCHUNK):
        row_base = wid * RPW + chunk * RC
        pltpu.sync_copy(idx_hbm.at[pl.ds(row_base, RC)], idx_v)
        scatter(ones, add=True)
        pltpu.sync_copy(hist_v, counts_hbm.at[pl.ds(row_base, RC)])
        scatter(zeros, add=False)


def _encode_kernel(counts_ref, w_ref, out_ref):
    out_ref[...] = jnp.dot(
        counts_ref[...], w_ref[0], preferred_element_type=jnp.float32
    ) * (1.0 / L)


def _score_kernel(a_ref, b_ref, out_ref):
    out_ref[...] = lax.dot_general(
        a_ref[...], b_ref[...], (((1,), (1,)), ((), ())),
        preferred_element_type=jnp.float32)


@jax.jit
def kernel(text_vec, cand_vecs, W_ctx, W_cand):
    idx = jnp.concatenate([text_vec, cand_vecs], axis=0).astype(jnp.int32)
    w = jnp.stack([
        jnp.pad(W_ctx, ((0, VP - V), (0, 0))),
        jnp.pad(W_cand, ((0, VP - V), (0, 0))),
    ])

    hist_fn = pl.kernel(
        _hist_kernel,
        out_type=jax.ShapeDtypeStruct((2 * B, VP), jnp.float32),
        mesh=plsc.VectorSubcoreMesh(
            core_axis_name="c", subcore_axis_name="s",
            num_cores=NC, num_subcores=NS),
        compiler_params=pltpu.CompilerParams(needs_layout_passes=False),
        scratch_types=[
            pltpu.VMEM((RC, L), jnp.int32),
            pltpu.VMEM((RC, VP), jnp.float32),
        ],
    )
    counts = hist_fn(idx)

    nblk = B // BB
    encs = pl.pallas_call(
        _encode_kernel,
        grid=(2 * nblk,),
        in_specs=[
            pl.BlockSpec((BB, VP), lambda i: (i, 0)),
            pl.BlockSpec((1, VP, D), lambda i: (i // nblk, 0, 0)),
        ],
        out_specs=pl.BlockSpec((BB, D), lambda i: (i, 0)),
        out_shape=jax.ShapeDtypeStruct((2 * B, D), jnp.float32),
    )(counts, w)

    ctx_enc, cand_enc = encs[:B], encs[B:]
    out = pl.pallas_call(
        _score_kernel,
        grid=(B // BM, B // BN),
        in_specs=[
            pl.BlockSpec((BM, D), lambda i, j: (i, 0)),
            pl.BlockSpec((BN, D), lambda i, j: (j, 0)),
        ],
        out_specs=pl.BlockSpec((BM, BN), lambda i, j: (i, j)),
        out_shape=jax.ShapeDtypeStruct((B, B), jnp.float32),
    )(ctx_enc, cand_enc)
    return out


# speedup vs baseline: 1.3950x; 1.3950x over previous
import functools

import jax
import jax.numpy as jnp
from jax import lax
from jax.experimental import pallas as pl
from jax.experimental.pallas import tpu as pltpu
from jax.experimental.pallas import tpu_sc as plsc

V, D, B, L = 1000, 64, 4096, 200
VP = 1024
NC, NS = 2, 16
NW = NC * NS
RPW = 2 * B // NW
RC = 64
NCHUNK = RPW // RC
BB = 256
BM = 1024
BN = 1024


NVEC = L // 16
TAIL = L - NVEC * 16


def _hist_kernel(idx_hbm, counts_hbm, idx_v, hist_v):
    wid = lax.axis_index("s") * NC + lax.axis_index("c")
    lane = lax.iota(jnp.int32, 16)
    ones = jnp.ones((16,), jnp.float32)
    zeros = jnp.zeros((16,), jnp.float32)
    tail_mask = lane < TAIL

    def zero_body(r, c):
        for u in range(VP // 16):
            hist_v[r, pl.ds(u * 16, 16)] = zeros
        return c

    def scatter_body(r, c):
        rsplat = jnp.full((16,), 0, jnp.int32) + r
        for j in range(NVEC):
            iv = idx_v[pl.ds(r * L + j * 16, 16)]
            plsc.addupdate_scatter(hist_v, [rsplat, iv], ones)
        iv = idx_v[pl.ds(r * L + NVEC * 16, 16)]
        plsc.addupdate_scatter(hist_v, [rsplat, iv], ones, mask=tail_mask)
        return c

    for chunk in range(NCHUNK):
        row_base = wid * RPW + chunk * RC
        pltpu.sync_copy(
            idx_hbm.at[pl.ds(row_base * L, RC * L)],
            idx_v.at[pl.ds(0, RC * L)])
        lax.fori_loop(0, RC, zero_body, 0)
        lax.fori_loop(0, RC, scatter_body, 0)
        pltpu.sync_copy(hist_v, counts_hbm.at[pl.ds(row_base, RC)])


def _encode_kernel(counts_ref, w_ref, out_ref):
    out_ref[...] = jnp.dot(
        counts_ref[...], w_ref[0], preferred_element_type=jnp.float32
    ) * (1.0 / L)


def _score_kernel(a_ref, b_ref, out_ref):
    out_ref[...] = lax.dot_general(
        a_ref[...], b_ref[...], (((1,), (1,)), ((), ())),
        preferred_element_type=jnp.float32)


@jax.jit
def kernel(text_vec, cand_vecs, W_ctx, W_cand):
    idx = jnp.concatenate([text_vec, cand_vecs], axis=0).astype(jnp.int32)
    w = jnp.stack([
        jnp.pad(W_ctx, ((0, VP - V), (0, 0))),
        jnp.pad(W_cand, ((0, VP - V), (0, 0))),
    ])

    hist_fn = pl.kernel(
        _hist_kernel,
        out_type=jax.ShapeDtypeStruct((2 * B, VP), jnp.float32),
        mesh=plsc.VectorSubcoreMesh(
            core_axis_name="c", subcore_axis_name="s",
            num_cores=NC, num_subcores=NS),
        compiler_params=pltpu.CompilerParams(needs_layout_passes=False),
        scratch_types=[
            pltpu.VMEM((RC * L + 16,), jnp.int32),
            pltpu.VMEM((RC, VP), jnp.float32),
        ],
    )
    counts = hist_fn(idx.reshape(-1))

    nblk = B // BB
    encs = pl.pallas_call(
        _encode_kernel,
        grid=(2 * nblk,),
        in_specs=[
            pl.BlockSpec((BB, VP), lambda i: (i, 0)),
            pl.BlockSpec((1, VP, D), lambda i: (i // nblk, 0, 0)),
        ],
        out_specs=pl.BlockSpec((BB, D), lambda i: (i, 0)),
        out_shape=jax.ShapeDtypeStruct((2 * B, D), jnp.float32),
    )(counts, w)

    ctx_enc, cand_enc = encs[:B], encs[B:]
    out = pl.pallas_call(
        _score_kernel,
        grid=(B // BM, B // BN),
        in_specs=[
            pl.BlockSpec((BM, D), lambda i, j: (i, 0)),
            pl.BlockSpec((BN, D), lambda i, j: (j, 0)),
        ],
        out_specs=pl.BlockSpec((BM, BN), lambda i, j: (i, j)),
        out_shape=jax.ShapeDtypeStruct((B, B), jnp.float32),
    )(ctx_enc, cand_enc)
    return out


# speedup vs baseline: 1.6667x; 1.1947x over previous
import functools

import jax
import jax.numpy as jnp
from jax import lax
from jax.experimental import pallas as pl
from jax.experimental.pallas import tpu as pltpu
from jax.experimental.pallas import tpu_sc as plsc

V, D, B, L = 1000, 64, 4096, 200
VP = 1024
NC, NS = 2, 16
NW = NC * NS
HROWS = B // NW
RC = 32
NCHUNK = HROWS // RC
BB = 256
BM = 1024
BN = 1024
NVEC = L // 16
TAIL = L - NVEC * 16


def _hist_kernel(text_hbm, cand_hbm, counts_hbm,
                 idx_v, hist0, hist1, sem0, sem1):
    wid = lax.axis_index("s") * NC + lax.axis_index("c")
    lane = lax.iota(jnp.int32, 16)
    ones = jnp.ones((16,), jnp.float32)
    zeros = jnp.zeros((16,), jnp.float32)
    tail_mask = lane >= 16 - TAIL
    hists = [hist0, hist1]
    sems = [sem0, sem1]

    def do_chunk(hist_v, sem, chunk, out_ref):
        def zero_body(r, c):
            for u in range(VP // 16):
                hist_v[r, pl.ds(u * 16, 16)] = zeros
            return c

        def scatter_body(r, c):
            rsplat = jnp.full((16,), 0, jnp.int32) + r
            ridx = chunk * RC + r
            for j in range(NVEC):
                iv = idx_v[ridx, pl.ds(j * 16, 16)]
                plsc.addupdate_scatter(hist_v, [rsplat, iv], ones)
            iv = idx_v[ridx, pl.ds(L - 16, 16)]
            plsc.addupdate_scatter(hist_v, [rsplat, iv], ones,
                                   mask=tail_mask)
            return c

        lax.fori_loop(0, RC, zero_body, 0)
        lax.fori_loop(0, RC, scatter_body, 0)
        return pltpu.async_copy(hist_v, out_ref, sem)

    pending = []
    for half, src_hbm in enumerate([text_hbm, cand_hbm]):
        pltpu.sync_copy(src_hbm.at[pl.ds(wid * HROWS, HROWS)], idx_v)
        for chunk in range(NCHUNK):
            b = (half * NCHUNK + chunk) % 2
            if len(pending) >= 2:
                pending.pop(0).wait()
            row_base = half * B + wid * HROWS + chunk * RC
            pending.append(do_chunk(
                hists[b], sems[b], chunk,
                counts_hbm.at[pl.ds(row_base, RC)]))
    for d in pending:
        d.wait()


def _encode_kernel(counts_ref, w_ref, out_ref):
    out_ref[...] = jnp.dot(
        counts_ref[...], w_ref[0], preferred_element_type=jnp.float32
    ) * (1.0 / L)


def _score_kernel(a_ref, b_ref, out_ref):
    out_ref[...] = lax.dot_general(
        a_ref[...], b_ref[...], (((1,), (1,)), ((), ())),
        preferred_element_type=jnp.float32)


@jax.jit
def kernel(text_vec, cand_vecs, W_ctx, W_cand):
    text_vec = text_vec.astype(jnp.int32)
    cand_vecs = cand_vecs.astype(jnp.int32)
    w = jnp.stack([
        jnp.pad(W_ctx, ((0, VP - V), (0, 0))),
        jnp.pad(W_cand, ((0, VP - V), (0, 0))),
    ])

    hist_fn = pl.kernel(
        _hist_kernel,
        out_type=jax.ShapeDtypeStruct((2 * B, VP), jnp.float32),
        mesh=plsc.VectorSubcoreMesh(
            core_axis_name="c", subcore_axis_name="s",
            num_cores=NC, num_subcores=NS),
        compiler_params=pltpu.CompilerParams(needs_layout_passes=False),
        scratch_types=[
            pltpu.VMEM((HROWS, L), jnp.int32),
            pltpu.VMEM((RC, VP), jnp.float32),
            pltpu.VMEM((RC, VP), jnp.float32),
            pltpu.SemaphoreType.DMA,
            pltpu.SemaphoreType.DMA,
        ],
    )
    counts = hist_fn(text_vec, cand_vecs)

    nblk = B // BB
    encs = pl.pallas_call(
        _encode_kernel,
        grid=(2 * nblk,),
        in_specs=[
            pl.BlockSpec((BB, VP), lambda i: (i, 0)),
            pl.BlockSpec((1, VP, D), lambda i: (i // nblk, 0, 0)),
        ],
        out_specs=pl.BlockSpec((BB, D), lambda i: (i, 0)),
        out_shape=jax.ShapeDtypeStruct((2 * B, D), jnp.float32),
    )(counts, w)

    ctx_enc, cand_enc = encs[:B], encs[B:]
    out = pl.pallas_call(
        _score_kernel,
        grid=(B // BM, B // BN),
        in_specs=[
            pl.BlockSpec((BM, D), lambda i, j: (i, 0)),
            pl.BlockSpec((BN, D), lambda i, j: (j, 0)),
        ],
        out_specs=pl.BlockSpec((BM, BN), lambda i, j: (i, j)),
        out_shape=jax.ShapeDtypeStruct((B, B), jnp.float32),
    )(ctx_enc, cand_enc)
    return out


# speedup vs baseline: 1.9637x; 1.1782x over previous
import functools

import jax
import jax.numpy as jnp
from jax import lax
from jax.experimental import pallas as pl
from jax.experimental.pallas import tpu as pltpu
from jax.experimental.pallas import tpu_sc as plsc

V, D, B, L = 1000, 64, 4096, 200
VP = 1024
NC, NS = 2, 16
NW = NC * NS
HROWS = B // NW
RC = 32
NCHUNK = HROWS // RC
BB = 256
BM = 1024
BN = 1024
NVEC = L // 16
TAIL = L - NVEC * 16


def _hist_kernel(text_hbm, cand_hbm, counts_hbm,
                 idx_v, hist0, hist1, sem0, sem1):
    wid = lax.axis_index("s") * NC + lax.axis_index("c")
    lane = lax.iota(jnp.int32, 16)
    ones = jnp.ones((16,), jnp.float32)
    zeros = jnp.zeros((16,), jnp.float32)
    tail_mask = lane >= 16 - TAIL
    hists = [hist0, hist1]
    sems = [sem0, sem1]

    def do_chunk(hist_v, sem, chunk, out_ref):
        def zero_body(r, c):
            for u in range(VP // 16):
                hist_v[r, pl.ds(u * 16, 16)] = zeros
            return c

        def scatter_body(r, c):
            rsplat = jnp.full((16,), 0, jnp.int32) + r
            ridx = chunk * RC + r
            for j in range(NVEC):
                iv = idx_v[ridx, pl.ds(j * 16, 16)]
                plsc.addupdate_scatter(hist_v, [rsplat, iv], ones)
            iv = idx_v[ridx, pl.ds(L - 16, 16)]
            plsc.addupdate_scatter(hist_v, [rsplat, iv], ones,
                                   mask=tail_mask)
            return c

        @plsc.parallel_loop(0, RC, unroll=2)
        def _(r):
            zero_body(r, 0)

        @plsc.parallel_loop(0, RC, unroll=2)
        def _(r):
            scatter_body(r, 0)

        return pltpu.async_copy(hist_v, out_ref, sem)

    pending = []
    for half, src_hbm in enumerate([text_hbm, cand_hbm]):
        pltpu.sync_copy(src_hbm.at[pl.ds(wid * HROWS, HROWS)], idx_v)
        for chunk in range(NCHUNK):
            b = (half * NCHUNK + chunk) % 2
            if len(pending) >= 2:
                pending.pop(0).wait()
            row_base = half * B + wid * HROWS + chunk * RC
            pending.append(do_chunk(
                hists[b], sems[b], chunk,
                counts_hbm.at[pl.ds(row_base, RC)]))
    for d in pending:
        d.wait()


def _encode_kernel(counts_ref, w_ref, out_ref):
    out_ref[...] = jnp.dot(
        counts_ref[...], w_ref[0], preferred_element_type=jnp.float32
    ) * (1.0 / L)


def _score_kernel(a_ref, b_ref, out_ref):
    out_ref[...] = lax.dot_general(
        a_ref[...], b_ref[...], (((1,), (1,)), ((), ())),
        preferred_element_type=jnp.float32)


@jax.jit
def kernel(text_vec, cand_vecs, W_ctx, W_cand):
    text_vec = text_vec.astype(jnp.int32)
    cand_vecs = cand_vecs.astype(jnp.int32)
    w = jnp.stack([
        jnp.pad(W_ctx, ((0, VP - V), (0, 0))),
        jnp.pad(W_cand, ((0, VP - V), (0, 0))),
    ])

    hist_fn = pl.kernel(
        _hist_kernel,
        out_type=jax.ShapeDtypeStruct((2 * B, VP), jnp.float32),
        mesh=plsc.VectorSubcoreMesh(
            core_axis_name="c", subcore_axis_name="s",
            num_cores=NC, num_subcores=NS),
        compiler_params=pltpu.CompilerParams(needs_layout_passes=False),
        scratch_types=[
            pltpu.VMEM((HROWS, L), jnp.int32),
            pltpu.VMEM((RC, VP), jnp.float32),
            pltpu.VMEM((RC, VP), jnp.float32),
            pltpu.SemaphoreType.DMA,
            pltpu.SemaphoreType.DMA,
        ],
    )
    counts = hist_fn(text_vec, cand_vecs)

    nblk = B // BB
    encs = pl.pallas_call(
        _encode_kernel,
        grid=(2 * nblk,),
        in_specs=[
            pl.BlockSpec((BB, VP), lambda i: (i, 0)),
            pl.BlockSpec((1, VP, D), lambda i: (i // nblk, 0, 0)),
        ],
        out_specs=pl.BlockSpec((BB, D), lambda i: (i, 0)),
        out_shape=jax.ShapeDtypeStruct((2 * B, D), jnp.float32),
    )(counts, w)

    ctx_enc, cand_enc = encs[:B], encs[B:]
    out = pl.pallas_call(
        _score_kernel,
        grid=(B // BM, B // BN),
        in_specs=[
            pl.BlockSpec((BM, D), lambda i, j: (i, 0)),
            pl.BlockSpec((BN, D), lambda i, j: (j, 0)),
        ],
        out_specs=pl.BlockSpec((BM, BN), lambda i, j: (i, j)),
        out_shape=jax.ShapeDtypeStruct((B, B), jnp.float32),
    )(ctx_enc, cand_enc)
    return out
